# initial kernel scaffold (unmeasured)
import jax
import jax.numpy as jnp
from jax import lax
from jax.experimental import pallas as pl
from jax.experimental.pallas import tpu as pltpu

N_DEV = 8


def _gemm(a, b):
    return lax.dot_general(
        a,
        b,
        (((1,), (0,)), ((), ())),
        precision=lax.Precision.HIGHEST,
        preferred_element_type=jnp.float32,
    )


def _snap_e4m3(t):
    a = jnp.abs(t)
    u = lax.bitcast_convert_type(a, jnp.int32)
    add = jnp.int32(0x7FFFF) + ((u >> 20) & 1)
    v = (u + add) & jnp.int32(-0x100000)
    normal = lax.bitcast_convert_type(v, jnp.float32)
    sub = jnp.round(a * 512.0) * (1.0 / 512.0)
    snapped = jnp.where(a >= 2.0**-6, normal, sub)
    snapped = jnp.minimum(snapped, 448.0)
    return jnp.where(t < 0.0, -snapped, snapped)


def kernel(x, w_mat):
    m_per, k = x.shape
    n = w_mat.shape[1]
    half = m_per // 2
    m_tot = N_DEV * m_per

    def body(
        x_ref,
        w_ref,
        out_ref,
        cw_ref,
        ccw_ref,
        amax_ref,
        cw_send,
        cw_recv,
        ccw_send,
        ccw_recv,
        ax_send,
        ax_recv,
    ):
        my = lax.axis_index("i")
        left = lax.rem(my + (N_DEV - 1), N_DEV)
        right = lax.rem(my + 1, N_DEV)

        barrier = pltpu.get_barrier_semaphore()
        for nbr in (left, right):
            pl.semaphore_signal(
                barrier,
                inc=1,
                device_id=(nbr,),
                device_id_type=pl.DeviceIdType.MESH,
            )
        pl.semaphore_wait(barrier, 2)

        amax = jnp.float32(0.0)

        def block_gemm(rows, row_start):
            res = _gemm(rows, w_ref[...])
            out_ref[pl.ds(row_start, res.shape[0]), :] = res
            return jnp.max(jnp.abs(res))

        for h in range(N_DEV - 1):
            s = h % 2
            cw_src = x_ref.at[0:half] if h == 0 else cw_ref.at[(h - 1) % 2]
            ccw_src = x_ref.at[half:m_per] if h == 0 else ccw_ref.at[(h - 1) % 2]
            cw = pltpu.make_async_remote_copy(
                src_ref=cw_src,
                dst_ref=cw_ref.at[s],
                send_sem=cw_send.at[s],
                recv_sem=cw_recv.at[s],
                device_id=(right,),
                device_id_type=pl.DeviceIdType.MESH,
            )
            ccw = pltpu.make_async_remote_copy(
                src_ref=ccw_src,
                dst_ref=ccw_ref.at[s],
                send_sem=ccw_send.at[s],
                recv_sem=ccw_recv.at[s],
                device_id=(left,),
                device_id_type=pl.DeviceIdType.MESH,
            )
            cw.start()
            ccw.start()
            if h == 0:
                amax = jnp.maximum(amax, block_gemm(x_ref[...], my * m_per))
            cw.wait()
            ccw.wait()
            o_cw = lax.rem(my + (N_DEV - 1 - h), N_DEV)
            o_ccw = lax.rem(my + (h + 1), N_DEV)
            amax = jnp.maximum(amax, block_gemm(cw_ref[s], o_cw * m_per))
            amax = jnp.maximum(
                amax, block_gemm(ccw_ref[s], o_ccw * m_per + half)
            )

        amax_ref[0, :, :] = jnp.full((8, 128), amax, jnp.float32)
        for h in range(N_DEV - 1):
            s = h % 2
            r = (h + 1) % 2
            hop = pltpu.make_async_remote_copy(
                src_ref=amax_ref.at[s],
                dst_ref=amax_ref.at[r],
                send_sem=ax_send.at[s],
                recv_sem=ax_recv.at[r],
                device_id=(right,),
                device_id_type=pl.DeviceIdType.MESH,
            )
            hop.start()
            hop.wait()
            amax = jnp.maximum(amax, amax_ref[r, 0, 0])
            amax_ref[s, :, :] = jnp.full((8, 128), amax_ref[r, 0, 0], jnp.float32)

        scale = amax * (1.0 / 448.0)
        t = out_ref[...] * (448.0 / amax)
        out_ref[...] = _snap_e4m3(t) * scale

    return pl.pallas_call(
        body,
        out_shape=jax.ShapeDtypeStruct((m_tot, n), jnp.float32),
        in_specs=[
            pl.BlockSpec(memory_space=pltpu.VMEM),
            pl.BlockSpec(memory_space=pltpu.VMEM),
        ],
        out_specs=pl.BlockSpec(memory_space=pltpu.VMEM),
        scratch_shapes=[
            pltpu.VMEM((2, half, k), jnp.float32),
            pltpu.VMEM((2, half, k), jnp.float32),
            pltpu.VMEM((2, 8, 128), jnp.float32),
            pltpu.SemaphoreType.DMA((2,)),
            pltpu.SemaphoreType.DMA((2,)),
            pltpu.SemaphoreType.DMA((2,)),
            pltpu.SemaphoreType.DMA((2,)),
            pltpu.SemaphoreType.DMA((2,)),
            pltpu.SemaphoreType.DMA((2,)),
        ],
        compiler_params=pltpu.CompilerParams(collective_id=0),
    )(x, w_mat)


# baseline (device time: 421177 ns/iter reference)
import jax
import jax.numpy as jnp
from jax import lax
from jax.experimental import pallas as pl
from jax.experimental.pallas import tpu as pltpu

N_DEV = 8


def _gemm(a, b):
    return lax.dot_general(
        a,
        b,
        (((1,), (0,)), ((), ())),
        precision=lax.Precision.DEFAULT,
        preferred_element_type=jnp.float32,
    )


def _snap_e4m3(t):
    a = jnp.abs(t)
    u = lax.bitcast_convert_type(a, jnp.int32)
    add = jnp.int32(0x7FFFF) + ((u >> 20) & 1)
    v = (u + add) & jnp.int32(-0x100000)
    normal = lax.bitcast_convert_type(v, jnp.float32)
    sub = jnp.round(a * 512.0) * (1.0 / 512.0)
    snapped = jnp.where(a >= 2.0**-6, normal, sub)
    snapped = jnp.minimum(snapped, 448.0)
    return jnp.where(t < 0.0, -snapped, snapped)


def kernel(x, w_mat):
    m_per, k = x.shape
    n = w_mat.shape[1]
    half = m_per // 2
    m_tot = N_DEV * m_per

    def body(
        x_ref,
        w_ref,
        out_ref,
        cw_ref,
        ccw_ref,
        amax_ref,
        cw_send,
        cw_recv,
        ccw_send,
        ccw_recv,
        ax_send,
        ax_recv,
    ):
        my = lax.axis_index("i")
        left = lax.rem(my + (N_DEV - 1), N_DEV)
        right = lax.rem(my + 1, N_DEV)

        barrier = pltpu.get_barrier_semaphore()
        for nbr in (left, right):
            pl.semaphore_signal(
                barrier,
                inc=1,
                device_id=(nbr,),
                device_id_type=pl.DeviceIdType.MESH,
            )
        pl.semaphore_wait(barrier, 2)

        def block_gemm(src_ref, nrows, row_start):
            out_ref[pl.ds(row_start, nrows), :] = _gemm(
                src_ref[...], w_ref[...]
            )

        for h in range(N_DEV - 1):
            s = h % 2
            cw_src = x_ref.at[0:half] if h == 0 else cw_ref.at[(h - 1) % 2]
            ccw_src = x_ref.at[half:m_per] if h == 0 else ccw_ref.at[(h - 1) % 2]
            cw = pltpu.make_async_remote_copy(
                src_ref=cw_src,
                dst_ref=cw_ref.at[s],
                send_sem=cw_send.at[s],
                recv_sem=cw_recv.at[s],
                device_id=(right,),
                device_id_type=pl.DeviceIdType.MESH,
            )
            ccw = pltpu.make_async_remote_copy(
                src_ref=ccw_src,
                dst_ref=ccw_ref.at[s],
                send_sem=ccw_send.at[s],
                recv_sem=ccw_recv.at[s],
                device_id=(left,),
                device_id_type=pl.DeviceIdType.MESH,
            )
            cw.start()
            ccw.start()
            if h == 0:
                block_gemm(x_ref, m_per, my * m_per)
            cw.wait()
            ccw.wait()
            o_cw = lax.rem(my + (N_DEV - 1 - h), N_DEV)
            o_ccw = lax.rem(my + (h + 1), N_DEV)
            block_gemm(cw_ref.at[s], half, o_cw * m_per)
            block_gemm(ccw_ref.at[s], half, o_ccw * m_per + half)

        def amax_blk(i, acc):
            return jnp.maximum(
                acc, jnp.max(jnp.abs(out_ref[pl.ds(i * 256, 256), :]))
            )

        amax = lax.fori_loop(0, m_tot // 256, amax_blk, jnp.float32(0.0))

        amax_ref[0, :, :] = jnp.full((8, 128), amax, jnp.float32)
        for h in range(N_DEV - 1):
            s = h % 2
            r = (h + 1) % 2
            hop = pltpu.make_async_remote_copy(
                src_ref=amax_ref.at[s],
                dst_ref=amax_ref.at[r],
                send_sem=ax_send.at[s],
                recv_sem=ax_recv.at[r],
                device_id=(right,),
                device_id_type=pl.DeviceIdType.MESH,
            )
            hop.start()
            hop.wait()
            amax = jnp.maximum(amax, amax_ref[r, 0, 0])

        scale = amax * (1.0 / 448.0)
        inv = 448.0 / amax
        blk = 128

        def quant_blk(i, carry):
            rs = i * blk
            t = out_ref[pl.ds(rs, blk), :] * inv
            out_ref[pl.ds(rs, blk), :] = _snap_e4m3(t) * scale
            return carry

        lax.fori_loop(0, m_tot // blk, quant_blk, 0)

    return pl.pallas_call(
        body,
        out_shape=jax.ShapeDtypeStruct((m_tot, n), jnp.float32),
        in_specs=[
            pl.BlockSpec(memory_space=pltpu.VMEM),
            pl.BlockSpec(memory_space=pltpu.VMEM),
        ],
        out_specs=pl.BlockSpec(memory_space=pltpu.VMEM),
        scratch_shapes=[
            pltpu.VMEM((2, half, k), jnp.float32),
            pltpu.VMEM((2, half, k), jnp.float32),
            pltpu.VMEM((2, 8, 128), jnp.float32),
            pltpu.SemaphoreType.DMA((2,)),
            pltpu.SemaphoreType.DMA((2,)),
            pltpu.SemaphoreType.DMA((2,)),
            pltpu.SemaphoreType.DMA((2,)),
            pltpu.SemaphoreType.DMA((2,)),
            pltpu.SemaphoreType.DMA((2,)),
        ],
        compiler_params=pltpu.CompilerParams(
            collective_id=0,
            vmem_limit_bytes=63 * 1024 * 1024,
        ),
    )(x, w_mat)


# device time: 390770 ns/iter; 1.0778x vs baseline; 1.0778x over previous
import jax
import jax.numpy as jnp
from jax import lax
from jax.experimental import pallas as pl
from jax.experimental.pallas import tpu as pltpu

N_DEV = 8


def _gemm(a, b):
    return lax.dot_general(
        a,
        b,
        (((1,), (0,)), ((), ())),
        precision=lax.Precision.DEFAULT,
        preferred_element_type=jnp.float32,
    )


def _snap_e4m3(t):
    a = jnp.abs(t)
    u = lax.bitcast_convert_type(a, jnp.int32)
    add = jnp.int32(0x7FFFF) + ((u >> 20) & 1)
    v = (u + add) & jnp.int32(-0x100000)
    normal = lax.bitcast_convert_type(v, jnp.float32)
    sub = jnp.round(a * 512.0) * (1.0 / 512.0)
    snapped = jnp.where(a >= 2.0**-6, normal, sub)
    snapped = jnp.minimum(snapped, 448.0)
    return jnp.where(t < 0.0, -snapped, snapped)


def kernel(x, w_mat):
    m_per, k = x.shape
    n = w_mat.shape[1]
    half = m_per // 2
    m_tot = N_DEV * m_per

    def body(
        x_ref,
        w_ref,
        out_ref,
        cw_ref,
        ccw_ref,
        amax_ref,
        cw_send,
        cw_recv,
        ccw_send,
        ccw_recv,
        ax_send,
        ax_recv,
        credit_cw,
        credit_ccw,
    ):
        my = lax.axis_index("i")
        left = lax.rem(my + (N_DEV - 1), N_DEV)
        right = lax.rem(my + 1, N_DEV)

        barrier = pltpu.get_barrier_semaphore()
        for nbr in (left, right):
            pl.semaphore_signal(
                barrier,
                inc=1,
                device_id=(nbr,),
                device_id_type=pl.DeviceIdType.MESH,
            )
        pl.semaphore_wait(barrier, 2)

        def block_gemm(src_ref, nrows, row_start):
            out_ref[pl.ds(row_start, nrows), :] = _gemm(
                src_ref[...], w_ref[...]
            )

        def make_hop(h, cw_src, ccw_src):
            s = h % 2
            cw = pltpu.make_async_remote_copy(
                src_ref=cw_src,
                dst_ref=cw_ref.at[s],
                send_sem=cw_send.at[s],
                recv_sem=cw_recv.at[s],
                device_id=(right,),
                device_id_type=pl.DeviceIdType.MESH,
            )
            ccw = pltpu.make_async_remote_copy(
                src_ref=ccw_src,
                dst_ref=ccw_ref.at[s],
                send_sem=ccw_send.at[s],
                recv_sem=ccw_recv.at[s],
                device_id=(left,),
                device_id_type=pl.DeviceIdType.MESH,
            )
            return cw, ccw

        hops = [None] * (N_DEV - 1)
        hops[0] = make_hop(0, x_ref.at[0:half], x_ref.at[half:m_per])
        hops[0][0].start()
        hops[0][1].start()
        block_gemm(x_ref, m_per, my * m_per)

        for h in range(N_DEV - 1):
            s = h % 2
            hops[h][0].wait_recv()
            hops[h][1].wait_recv()
            hops[h][0].wait_send()
            hops[h][1].wait_send()
            if h < N_DEV - 2:
                hops[h + 1] = make_hop(
                    h + 1, cw_ref.at[s], ccw_ref.at[s]
                )
                hops[h + 1][0].start()
                hops[h + 1][1].start()
            o_cw = lax.rem(my + (N_DEV - 1 - h), N_DEV)
            o_ccw = lax.rem(my + (h + 1), N_DEV)
            block_gemm(cw_ref.at[s], half, o_cw * m_per)
            block_gemm(ccw_ref.at[s], half, o_ccw * m_per + half)

        def amax_blk(i, acc):
            return jnp.maximum(
                acc, jnp.max(jnp.abs(out_ref[pl.ds(i * 256, 256), :]))
            )

        amax = lax.fori_loop(0, m_tot // 256, amax_blk, jnp.float32(0.0))

        amax_ref[0, :, :] = jnp.full((8, 128), amax, jnp.float32)
        for h in range(N_DEV - 1):
            s = h % 2
            r = (h + 1) % 2
            hop = pltpu.make_async_remote_copy(
                src_ref=amax_ref.at[s],
                dst_ref=amax_ref.at[r],
                send_sem=ax_send.at[s],
                recv_sem=ax_recv.at[r],
                device_id=(right,),
                device_id_type=pl.DeviceIdType.MESH,
            )
            hop.start()
            hop.wait()
            amax = jnp.maximum(amax, amax_ref[r, 0, 0])

        scale = amax * (1.0 / 448.0)
        inv = 448.0 / amax
        blk = 128

        def quant_blk(i, carry):
            rs = i * blk
            t = out_ref[pl.ds(rs, blk), :] * inv
            out_ref[pl.ds(rs, blk), :] = _snap_e4m3(t) * scale
            return carry

        lax.fori_loop(0, m_tot // blk, quant_blk, 0)

    return pl.pallas_call(
        body,
        out_shape=jax.ShapeDtypeStruct((m_tot, n), jnp.float32),
        in_specs=[
            pl.BlockSpec(memory_space=pltpu.VMEM),
            pl.BlockSpec(memory_space=pltpu.VMEM),
        ],
        out_specs=pl.BlockSpec(memory_space=pltpu.VMEM),
        scratch_shapes=[
            pltpu.VMEM((2, half, k), jnp.float32),
            pltpu.VMEM((2, half, k), jnp.float32),
            pltpu.VMEM((2, 8, 128), jnp.float32),
            pltpu.SemaphoreType.DMA((2,)),
            pltpu.SemaphoreType.DMA((2,)),
            pltpu.SemaphoreType.DMA((2,)),
            pltpu.SemaphoreType.DMA((2,)),
            pltpu.SemaphoreType.DMA((2,)),
            pltpu.SemaphoreType.DMA((2,)),
            pltpu.SemaphoreType.REGULAR,
            pltpu.SemaphoreType.REGULAR,
        ],
        compiler_params=pltpu.CompilerParams(
            collective_id=0,
            vmem_limit_bytes=63 * 1024 * 1024,
        ),
    )(x, w_mat)


# device time: 373607 ns/iter; 1.1273x vs baseline; 1.0459x over previous
import jax
import jax.numpy as jnp
from jax import lax
from jax.experimental import pallas as pl
from jax.experimental.pallas import tpu as pltpu

N_DEV = 8


def _gemm(a, b):
    return lax.dot_general(
        a,
        b,
        (((1,), (0,)), ((), ())),
        precision=lax.Precision.DEFAULT,
        preferred_element_type=jnp.float32,
    )


def _snap_e4m3(t):
    a = jnp.abs(t)
    u = lax.bitcast_convert_type(a, jnp.int32)
    add = jnp.int32(0x7FFFF) + ((u >> 20) & 1)
    v = (u + add) & jnp.int32(-0x100000)
    normal = lax.bitcast_convert_type(v, jnp.float32)
    sub = jnp.round(a * 512.0) * (1.0 / 512.0)
    snapped = jnp.where(a >= 2.0**-6, normal, sub)
    snapped = jnp.minimum(snapped, 448.0)
    return jnp.where(t < 0.0, -snapped, snapped)


def kernel(x, w_mat):
    m_per, k = x.shape
    n = w_mat.shape[1]
    half = m_per // 2
    m_tot = N_DEV * m_per

    def body(
        x_ref,
        w_ref,
        out_ref,
        cw_ref,
        ccw_ref,
        amax_ref,
        cw_send,
        cw_recv,
        ccw_send,
        ccw_recv,
        ax_send,
        ax_recv,
        credit_cw,
        credit_ccw,
    ):
        my = lax.axis_index("i")
        left = lax.rem(my + (N_DEV - 1), N_DEV)
        right = lax.rem(my + 1, N_DEV)

        barrier = pltpu.get_barrier_semaphore()
        for nbr in (left, right):
            pl.semaphore_signal(
                barrier,
                inc=1,
                device_id=(nbr,),
                device_id_type=pl.DeviceIdType.MESH,
            )
        pl.semaphore_wait(barrier, 2)

        def block_gemm(src_ref, nrows, row_start):
            out_ref[pl.ds(row_start, nrows), :] = _gemm(
                src_ref[...], w_ref[...]
            )

        def make_hop(h, cw_src, ccw_src):
            s = h % 2
            cw = pltpu.make_async_remote_copy(
                src_ref=cw_src,
                dst_ref=cw_ref.at[s],
                send_sem=cw_send.at[s],
                recv_sem=cw_recv.at[s],
                device_id=(right,),
                device_id_type=pl.DeviceIdType.MESH,
            )
            ccw = pltpu.make_async_remote_copy(
                src_ref=ccw_src,
                dst_ref=ccw_ref.at[s],
                send_sem=ccw_send.at[s],
                recv_sem=ccw_recv.at[s],
                device_id=(left,),
                device_id_type=pl.DeviceIdType.MESH,
            )
            return cw, ccw

        hops = [None] * (N_DEV - 1)
        hops[0] = make_hop(0, x_ref.at[0:half], x_ref.at[half:m_per])
        hops[0][0].start()
        hops[0][1].start()
        block_gemm(x_ref, m_per, my * m_per)

        for h in range(N_DEV - 1):
            s = h % 2
            hops[h][0].wait_recv()
            hops[h][1].wait_recv()
            hops[h][0].wait_send()
            hops[h][1].wait_send()
            if h < N_DEV - 2:
                hops[h + 1] = make_hop(
                    h + 1, cw_ref.at[s], ccw_ref.at[s]
                )
                hops[h + 1][0].start()
                hops[h + 1][1].start()
            o_cw = lax.rem(my + (N_DEV - 1 - h), N_DEV)
            o_ccw = lax.rem(my + (h + 1), N_DEV)
            block_gemm(cw_ref.at[s], half, o_cw * m_per)
            block_gemm(ccw_ref.at[s], half, o_ccw * m_per + half)

        def amax_blk(i, acc):
            return jnp.maximum(
                acc, jnp.max(jnp.abs(out_ref[pl.ds(i * 256, 256), :]))
            )

        amax = lax.fori_loop(0, m_tot // 256, amax_blk, jnp.float32(0.0))

        amax_ref[0, :, :] = jnp.full((8, 128), amax, jnp.float32)
        ax = [None] * N_DEV
        for d in range(1, N_DEV):
            ax[d] = pltpu.make_async_remote_copy(
                src_ref=amax_ref.at[0],
                dst_ref=amax_ref.at[d],
                send_sem=ax_send.at[d],
                recv_sem=ax_recv.at[d],
                device_id=(lax.rem(my + d, N_DEV),),
                device_id_type=pl.DeviceIdType.MESH,
            )
            ax[d].start()
        for d in range(1, N_DEV):
            ax[d].wait_recv()
            amax = jnp.maximum(amax, amax_ref[d, 0, 0])
        for d in range(1, N_DEV):
            ax[d].wait_send()

        scale = amax * (1.0 / 448.0)
        inv = 448.0 / amax
        blk = 128

        def quant_blk(i, carry):
            rs = i * blk
            t = out_ref[pl.ds(rs, blk), :] * inv
            q = t.astype(jnp.float8_e4m3fn).astype(jnp.float32)
            out_ref[pl.ds(rs, blk), :] = q * scale
            return carry

        lax.fori_loop(0, m_tot // blk, quant_blk, 0)

    return pl.pallas_call(
        body,
        out_shape=jax.ShapeDtypeStruct((m_tot, n), jnp.float32),
        in_specs=[
            pl.BlockSpec(memory_space=pltpu.VMEM),
            pl.BlockSpec(memory_space=pltpu.VMEM),
        ],
        out_specs=pl.BlockSpec(memory_space=pltpu.VMEM),
        scratch_shapes=[
            pltpu.VMEM((2, half, k), jnp.float32),
            pltpu.VMEM((2, half, k), jnp.float32),
            pltpu.VMEM((N_DEV, 8, 128), jnp.float32),
            pltpu.SemaphoreType.DMA((2,)),
            pltpu.SemaphoreType.DMA((2,)),
            pltpu.SemaphoreType.DMA((2,)),
            pltpu.SemaphoreType.DMA((2,)),
            pltpu.SemaphoreType.DMA((N_DEV,)),
            pltpu.SemaphoreType.DMA((N_DEV,)),
            pltpu.SemaphoreType.REGULAR,
            pltpu.SemaphoreType.REGULAR,
        ],
        compiler_params=pltpu.CompilerParams(
            collective_id=0,
            vmem_limit_bytes=63 * 1024 * 1024,
        ),
    )(x, w_mat)


# device time: 217106 ns/iter; 1.9400x vs baseline; 1.7209x over previous
import jax
import jax.numpy as jnp
from jax import lax
from jax.experimental import pallas as pl
from jax.experimental.pallas import tpu as pltpu

N_DEV = 8


def _gemm(a, b):
    return lax.dot_general(
        a,
        b,
        (((1,), (0,)), ((), ())),
        precision=lax.Precision.DEFAULT,
        preferred_element_type=jnp.float32,
    )


def _snap_e4m3(t):
    a = jnp.abs(t)
    u = lax.bitcast_convert_type(a, jnp.int32)
    add = jnp.int32(0x7FFFF) + ((u >> 20) & 1)
    v = (u + add) & jnp.int32(-0x100000)
    normal = lax.bitcast_convert_type(v, jnp.float32)
    sub = jnp.round(a * 512.0) * (1.0 / 512.0)
    snapped = jnp.where(a >= 2.0**-6, normal, sub)
    snapped = jnp.minimum(snapped, 448.0)
    return jnp.where(t < 0.0, -snapped, snapped)


def kernel(x, w_mat):
    m_per, k = x.shape
    n = w_mat.shape[1]
    half = m_per // 2
    m_tot = N_DEV * m_per

    def body(
        x_ref,
        w_ref,
        out_ref,
        xbf_ref,
        cw_ref,
        ccw_ref,
        amax_ref,
        cw_send,
        cw_recv,
        ccw_send,
        ccw_recv,
        ax_send,
        ax_recv,
        credit_cw,
        credit_ccw,
    ):
        my = lax.axis_index("i")
        left = lax.rem(my + (N_DEV - 1), N_DEV)
        right = lax.rem(my + 1, N_DEV)

        barrier = pltpu.get_barrier_semaphore()
        for nbr in (left, right):
            pl.semaphore_signal(
                barrier,
                inc=1,
                device_id=(nbr,),
                device_id_type=pl.DeviceIdType.MESH,
            )
        pl.semaphore_wait(barrier, 2)

        def block_gemm(src_ref, nrows, row_start):
            out_ref[pl.ds(row_start, nrows), :] = _gemm(
                src_ref[...], w_ref[...]
            )

        def make_hop(h, cw_src, ccw_src):
            s = h % 2
            cw = pltpu.make_async_remote_copy(
                src_ref=cw_src,
                dst_ref=cw_ref.at[s],
                send_sem=cw_send.at[s],
                recv_sem=cw_recv.at[s],
                device_id=(right,),
                device_id_type=pl.DeviceIdType.MESH,
            )
            ccw = pltpu.make_async_remote_copy(
                src_ref=ccw_src,
                dst_ref=ccw_ref.at[s],
                send_sem=ccw_send.at[s],
                recv_sem=ccw_recv.at[s],
                device_id=(left,),
                device_id_type=pl.DeviceIdType.MESH,
            )
            return cw, ccw

        def conv_blk(i, carry):
            rs = pl.ds(i * 128, 128)
            xbf_ref[rs, :] = x_ref[rs, :].astype(jnp.bfloat16)
            return carry

        lax.fori_loop(0, m_per // 128, conv_blk, 0)

        hops = [None] * (N_DEV - 1)
        hops[0] = make_hop(0, xbf_ref.at[0:half], xbf_ref.at[half:m_per])
        hops[0][0].start()
        hops[0][1].start()
        block_gemm(xbf_ref, m_per, my * m_per)

        for h in range(N_DEV - 1):
            s = h % 2
            hops[h][0].wait_recv()
            hops[h][1].wait_recv()
            hops[h][0].wait_send()
            hops[h][1].wait_send()
            if h < N_DEV - 2:
                hops[h + 1] = make_hop(
                    h + 1, cw_ref.at[s], ccw_ref.at[s]
                )
                hops[h + 1][0].start()
                hops[h + 1][1].start()
            o_cw = lax.rem(my + (N_DEV - 1 - h), N_DEV)
            o_ccw = lax.rem(my + (h + 1), N_DEV)
            block_gemm(cw_ref.at[s], half, o_cw * m_per)
            block_gemm(ccw_ref.at[s], half, o_ccw * m_per + half)

        def amax_blk(i, acc):
            return jnp.maximum(
                acc, jnp.max(jnp.abs(out_ref[pl.ds(i * 256, 256), :]))
            )

        amax = lax.fori_loop(0, m_tot // 256, amax_blk, jnp.float32(0.0))

        amax_ref[0, :, :] = jnp.full((8, 128), amax, jnp.float32)
        ax = [None] * N_DEV
        for d in range(1, N_DEV):
            ax[d] = pltpu.make_async_remote_copy(
                src_ref=amax_ref.at[0],
                dst_ref=amax_ref.at[d],
                send_sem=ax_send.at[d],
                recv_sem=ax_recv.at[d],
                device_id=(lax.rem(my + d, N_DEV),),
                device_id_type=pl.DeviceIdType.MESH,
            )
            ax[d].start()
        for d in range(1, N_DEV):
            ax[d].wait_recv()
            amax = jnp.maximum(amax, amax_ref[d, 0, 0])
        for d in range(1, N_DEV):
            ax[d].wait_send()

        scale = amax * (1.0 / 448.0)
        inv = 448.0 / amax
        blk = 128

        def quant_blk(i, carry):
            rs = i * blk
            t = out_ref[pl.ds(rs, blk), :] * inv
            q = t.astype(jnp.float8_e4m3fn).astype(jnp.float32)
            out_ref[pl.ds(rs, blk), :] = q * scale
            return carry

        lax.fori_loop(0, m_tot // blk, quant_blk, 0)

    return pl.pallas_call(
        body,
        out_shape=jax.ShapeDtypeStruct((m_tot, n), jnp.float32),
        in_specs=[
            pl.BlockSpec(memory_space=pltpu.VMEM),
            pl.BlockSpec(memory_space=pltpu.VMEM),
        ],
        out_specs=pl.BlockSpec(memory_space=pltpu.VMEM),
        scratch_shapes=[
            pltpu.VMEM((m_per, k), jnp.bfloat16),
            pltpu.VMEM((2, half, k), jnp.bfloat16),
            pltpu.VMEM((2, half, k), jnp.bfloat16),
            pltpu.VMEM((N_DEV, 8, 128), jnp.float32),
            pltpu.SemaphoreType.DMA((2,)),
            pltpu.SemaphoreType.DMA((2,)),
            pltpu.SemaphoreType.DMA((2,)),
            pltpu.SemaphoreType.DMA((2,)),
            pltpu.SemaphoreType.DMA((N_DEV,)),
            pltpu.SemaphoreType.DMA((N_DEV,)),
            pltpu.SemaphoreType.REGULAR,
            pltpu.SemaphoreType.REGULAR,
        ],
        compiler_params=pltpu.CompilerParams(
            collective_id=0,
            vmem_limit_bytes=63 * 1024 * 1024,
        ),
    )(x, w_mat)
